# Initial kernel scaffold; baseline (speedup 1.0000x reference)
import jax
import jax.numpy as jnp
from jax.experimental import pallas as pl


def kernel(feat1, feat2, feat, edge_index1, edge_index2, edge_index, W1, b1, W2, b2):
    raise NotImplementedError("write your pallas kernel here")


# speedup vs baseline: 3.5168x; 3.5168x over previous
import functools

import jax
import jax.numpy as jnp
from jax import lax
from jax.experimental import pallas as pl
from jax.experimental.pallas import tpu as pltpu
from jax.experimental.pallas import tpu_sc as plsc

N = 10000
NPAD = 10240
D = 128
E = 320000
NV = 3
NC = 2
NS = 16
NW = NC * NS
RPT = NPAD // NS
EPT = E // NW
ECD = 2000
EC = 64
NCHUNK = EPT // EC
ETAIL = EPT - NCHUNK * EC
BR = 1024

_MESH = plsc.VectorSubcoreMesh(
    core_axis_name="c", subcore_axis_name="s", num_cores=NC, num_subcores=NS
)


def _deg_body(es0, es1, es2, ed0, ed1, ed2, out, hist, idxb):
    earr = [es0, es1, es2, ed0, ed1, ed2]
    c = lax.axis_index("c")
    s = lax.axis_index("s")
    wid = c * NS + s
    base_e = wid * EPT
    zero16 = jnp.zeros((16,), jnp.float32)

    for a in range(6):
        def zloop(r, _):
            hist[pl.ds(r * 16, 16)] = zero16
            return 0
        lax.fori_loop(0, NPAD // 16, zloop, 0)

        ea = earr[a]

        def chunk(k, _):
            pltpu.sync_copy(ea.at[pl.ds(base_e + k * ECD, ECD)], idxb)

            def vec(j, _):
                vi = idxb[pl.ds(j * 16, 16)]
                cnt, lastm = plsc.scan_count(vi)
                plsc.addupdate_scatter(hist, [vi], cnt.astype(jnp.float32),
                                       mask=lastm)
                return 0
            lax.fori_loop(0, ECD // 16, vec, 0)
            return 0
        lax.fori_loop(0, EPT // ECD, chunk, 0)

        pltpu.sync_copy(hist, out.at[wid, a])


_deg_call = functools.partial(
    pl.kernel,
    out_type=jax.ShapeDtypeStruct((NW, 6, NPAD), jnp.float32),
    mesh=_MESH,
    scratch_types=[
        pltpu.VMEM((NPAD,), jnp.float32),
        pltpu.VMEM((ECD,), jnp.int32),
    ],
    compiler_params=pltpu.CompilerParams(needs_layout_passes=False),
)(_deg_body)


def _agg_body(hs, es0, es1, es2, ed0, ed1, ed2, out,
              agg, rows, idxs, idxd, idxst, idxdt):
    esrc = [es0, es1, es2]
    edst = [ed0, ed1, ed2]
    c = lax.axis_index("c")
    s = lax.axis_index("s")
    base_e = (c * NS + s) * EPT
    rbase = s * RPT
    zero16 = jnp.zeros((16,), jnp.float32)

    for v in range(NV):
        def zfill(r, _):
            for j in range(D // 16):
                rows[r, pl.ds(j * 16, 16)] = zero16
            return 0
        lax.fori_loop(0, EC, zfill, 0)
        for j in range(RPT // EC):
            pltpu.sync_copy(rows, agg.at[pl.ds(rbase + j * EC, EC), :])
        plsc.subcore_barrier()

        hv = hs.at[v]
        ea_s = esrc[v]
        ea_d = edst[v]

        def chunk(i, _):
            eb = base_e + i * EC
            pltpu.sync_copy(ea_s.at[pl.ds(eb, EC)], idxs)
            pltpu.sync_copy(ea_d.at[pl.ds(eb, EC)], idxd)
            pltpu.sync_copy(hv.at[idxs], rows)
            pltpu.sync_copy(rows, agg.at[idxd], add=True)
            return 0
        lax.fori_loop(0, NCHUNK, chunk, 0)

        eb = base_e + NCHUNK * EC
        pltpu.sync_copy(ea_s.at[pl.ds(eb, ETAIL)], idxst)
        pltpu.sync_copy(ea_d.at[pl.ds(eb, ETAIL)], idxdt)
        pltpu.sync_copy(hv.at[idxst], rows.at[pl.ds(0, ETAIL), :])
        pltpu.sync_copy(rows.at[pl.ds(0, ETAIL), :], agg.at[idxdt], add=True)
        plsc.subcore_barrier()

        for j in range(RPT // EC):
            pltpu.sync_copy(agg.at[pl.ds(rbase + j * EC, EC), :],
                            out.at[c, v, pl.ds(rbase + j * EC, EC), :])


_agg_call = functools.partial(
    pl.kernel,
    out_type=jax.ShapeDtypeStruct((NC, NV, NPAD, D), jnp.float32),
    mesh=_MESH,
    scratch_types=[
        pltpu.VMEM_SHARED((NPAD, D), jnp.float32),
        pltpu.VMEM((EC, D), jnp.float32),
        pltpu.VMEM((EC,), jnp.int32),
        pltpu.VMEM((EC,), jnp.int32),
        pltpu.VMEM((ETAIL,), jnp.int32),
        pltpu.VMEM((ETAIL,), jnp.int32),
    ],
)(_agg_body)


def _scales_body(p_ref, o_ref):
    dg = jnp.sum(p_ref[...], axis=0)
    o_ref[...] = lax.rsqrt(jnp.maximum(dg, 1.0))


def _tcb_body(x_ref, so_ref, w_ref, o_ref):
    o_ref[0] = jnp.dot(x_ref[0] * so_ref[0], w_ref[...],
                       preferred_element_type=jnp.float32)


def _tcd_body(p_ref, si_ref, so_ref, b_ref, w_ref, o_ref):
    agg = p_ref[0, 0] + p_ref[1, 0]
    z = agg * si_ref[0] + b_ref[...]
    act = jnp.maximum(z, 0.0)
    o_ref[0] = jnp.dot(act * so_ref[0], w_ref[...],
                       preferred_element_type=jnp.float32)


def _tcf_body(p_ref, si_ref, b_ref, o_ref):
    o_ref[0] = (p_ref[0, 0] + p_ref[1, 0]) * si_ref[0] + b_ref[...]


_GRID = (NV, NPAD // BR)
_spec_x = pl.BlockSpec((1, BR, D), lambda v, i: (v, i, 0))
_spec_s = pl.BlockSpec((1, BR, 1), lambda v, i: (v, i, 0))
_spec_w = pl.BlockSpec((D, D), lambda v, i: (0, 0))
_spec_b = pl.BlockSpec((1, D), lambda v, i: (0, 0))
_spec_p = pl.BlockSpec((NC, 1, BR, D), lambda v, i: (0, v, i, 0))
_out_sds = jax.ShapeDtypeStruct((NV, NPAD, D), jnp.float32)


def _scales_call(p):
    return pl.pallas_call(
        _scales_body, grid=(NPAD // BR,),
        in_specs=[pl.BlockSpec((NW, 6, BR), lambda i: (0, 0, i))],
        out_specs=pl.BlockSpec((6, BR), lambda i: (0, i)),
        out_shape=jax.ShapeDtypeStruct((6, NPAD), jnp.float32),
    )(p)


def _tcb_call(x, so, w):
    return pl.pallas_call(
        _tcb_body, grid=_GRID,
        in_specs=[_spec_x, _spec_s, _spec_w],
        out_specs=_spec_x, out_shape=_out_sds,
    )(x, so, w)


def _tcd_call(p, si, so, b, w):
    return pl.pallas_call(
        _tcd_body, grid=_GRID,
        in_specs=[_spec_p, _spec_s, _spec_s, _spec_b, _spec_w],
        out_specs=_spec_x, out_shape=_out_sds,
    )(p, si, so, b, w)


def _tcf_call(p, si, b):
    return pl.pallas_call(
        _tcf_body, grid=_GRID,
        in_specs=[_spec_p, _spec_s, _spec_b],
        out_specs=_spec_x, out_shape=_out_sds,
    )(p, si, b)


def kernel(feat1, feat2, feat, edge_index1, edge_index2, edge_index,
           W1, b1, W2, b2):
    feats = jnp.stack([feat1, feat2, feat])
    featsp = jnp.zeros((NV, NPAD, D), jnp.float32).at[:, :N, :].set(feats)
    e1 = edge_index1.astype(jnp.int32)
    e2 = edge_index2.astype(jnp.int32)
    e3 = edge_index.astype(jnp.int32)
    erows = (e1[0], e2[0], e3[0], e1[1], e2[1], e3[1])

    pdeg = _deg_call(*erows)
    scales = _scales_call(pdeg)
    s_out = scales[0:3].reshape(NV, NPAD, 1)
    s_in = scales[3:6].reshape(NV, NPAD, 1)

    h1 = _tcb_call(featsp, s_out, W1)
    p1 = _agg_call(h1, *erows)
    h2 = _tcd_call(p1, s_in, s_out, b1.reshape(1, D), W2)
    p2 = _agg_call(h2, *erows)
    z3 = _tcf_call(p2, s_in, b2.reshape(1, D))
    return (z3[0, :N], z3[1, :N], z3[2, :N])


# speedup vs baseline: 5.6332x; 1.6018x over previous
import functools

import jax
import jax.numpy as jnp
from jax import lax
from jax.experimental import pallas as pl
from jax.experimental.pallas import tpu as pltpu
from jax.experimental.pallas import tpu_sc as plsc

N = 10000
NPAD = 10112
D = 128
E = 320000
NV = 3
NC = 2
NS = 16
NW = NC * NS
RPT = NPAD // NS
EPT = E // NW
ECD1 = 1040
ECD2 = 640
EC = 48
NCHUNK = EPT // EC
ETAIL = EPT - NCHUNK * EC
WB = 48
BR = 1264

_MESH = plsc.VectorSubcoreMesh(
    core_axis_name="c", subcore_axis_name="s", num_cores=NC, num_subcores=NS
)


def _deg_body(es0, es1, es2, ed0, ed1, ed2, out, hist, idxb):
    earr = [es0, es1, es2, ed0, ed1, ed2]
    c = lax.axis_index("c")
    s = lax.axis_index("s")
    wid = c * NS + s
    base_e = wid * EPT
    zero16 = jnp.zeros((16,), jnp.float32)

    def count_chunk(nvec):
        def vec(j, _):
            vi = idxb[pl.ds(j * 16, 16)]
            cnt, lastm = plsc.scan_count(vi)
            plsc.addupdate_scatter(hist, [vi], cnt.astype(jnp.float32),
                                   mask=lastm)
            return 0
        lax.fori_loop(0, nvec, vec, 0)

    for a in range(6):
        def zloop(r, _):
            hist[pl.ds(r * 16, 16)] = zero16
            return 0
        lax.fori_loop(0, NPAD // 16, zloop, 0)

        ea = earr[a]

        def chunk(k, _):
            pltpu.sync_copy(ea.at[pl.ds(base_e + k * ECD1, ECD1)],
                            idxb.at[pl.ds(0, ECD1)])
            count_chunk(ECD1 // 16)
            return 0
        lax.fori_loop(0, 9, chunk, 0)

        pltpu.sync_copy(ea.at[pl.ds(base_e + 9 * ECD1, ECD2)],
                        idxb.at[pl.ds(0, ECD2)])
        count_chunk(ECD2 // 16)

        pltpu.sync_copy(hist, out.at[wid, a])


_deg_call = functools.partial(
    pl.kernel,
    out_type=jax.ShapeDtypeStruct((NW, 6, NPAD), jnp.float32),
    mesh=_MESH,
    scratch_types=[
        pltpu.VMEM((NPAD,), jnp.float32),
        pltpu.VMEM((ECD1,), jnp.int32),
    ],
    compiler_params=pltpu.CompilerParams(needs_layout_passes=False),
)(_deg_body)


def _agg_body(hs, es0, es1, es2, ed0, ed1, ed2, out,
              rows0, rows1, ixs0, ixs1, ixs2, ixs3, ixd0, ixd1, ixd2, ixd3,
              ixts, ixtd, agg,
              sg0, sg1, ss0, ss1, si0, si1, si2, si3, sw):
    esrc = [es0, es1, es2]
    edst = [ed0, ed1, ed2]
    rows = [rows0, rows1]
    ixs = [ixs0, ixs1, ixs2, ixs3]
    ixd = [ixd0, ixd1, ixd2, ixd3]
    sg = [sg0, sg1]
    ss = [ss0, ss1]
    si = [si0, si1, si2, si3]
    c = lax.axis_index("c")
    s = lax.axis_index("s")
    base_e = (c * NS + s) * EPT
    rbase = s * RPT
    zero16 = jnp.zeros((16,), jnp.float32)

    for v in range(NV):
        ea_s = esrc[v]
        ea_d = edst[v]
        hv = hs.at[v]

        def istart(ci, sl):
            pltpu.async_copy(ea_s.at[pl.ds(base_e + ci * EC, EC)],
                             ixs[sl], si[sl])
            pltpu.async_copy(ea_d.at[pl.ds(base_e + ci * EC, EC)],
                             ixd[sl], si[sl])

        def iwait(sl):
            pltpu.make_async_copy(ea_s.at[pl.ds(base_e, EC)],
                                  ixs[sl], si[sl]).wait()
            pltpu.make_async_copy(ea_d.at[pl.ds(base_e, EC)],
                                  ixd[sl], si[sl]).wait()

        def gstart(sl, b):
            pltpu.async_copy(hv.at[ixs[sl]], rows[b], sg[b])

        def gwait(sl, b):
            pltpu.make_async_copy(hv.at[ixs[sl]], rows[b], sg[b]).wait()

        def sstart(sl, b):
            pltpu.async_copy(rows[b], agg.at[ixd[sl]], ss[b], add=True)

        def swait(sl, b):
            pltpu.make_async_copy(rows[b], agg.at[ixd[sl]], ss[b]).wait()

        def zfill(r, _):
            for j in range(D // 16):
                rows0[r, pl.ds(j * 16, 16)] = zero16
            return 0
        lax.fori_loop(0, WB, zfill, 0)
        for j in range(RPT // WB):
            pltpu.async_copy(rows0, agg.at[pl.ds(rbase + j * WB, WB), :], sw)
        pltpu.async_copy(rows0.at[pl.ds(0, RPT % WB), :],
                         agg.at[pl.ds(rbase + (RPT // WB) * WB, RPT % WB), :],
                         sw)
        for j in range(RPT // WB):
            pltpu.make_async_copy(rows0,
                                  agg.at[pl.ds(rbase, WB), :], sw).wait()
        pltpu.make_async_copy(rows0.at[pl.ds(0, RPT % WB), :],
                              agg.at[pl.ds(rbase, RPT % WB), :], sw).wait()
        plsc.subcore_barrier()

        istart(0, 0)
        istart(1, 1)
        istart(2, 2)
        iwait(0)
        gstart(0, 0)

        def quad(i4, _):
            i0 = i4 * 4
            for b in range(4):
                i = i0 + b
                rb = b % 2
                gwait(b, rb)
                sstart(b, rb)

                @pl.when(i + 1 < NCHUNK)
                def _():
                    iwait((b + 1) % 4)

                    @pl.when(i >= 1)
                    def _():
                        swait((b + 3) % 4, (b + 1) % 2)
                    gstart((b + 1) % 4, (b + 1) % 2)

                    @pl.when(i + 3 < NCHUNK)
                    def _():
                        istart(i + 3, (b + 3) % 4)
            return 0
        lax.fori_loop(0, NCHUNK // 4, quad, 0)
        swait(2, 0)
        swait(3, 1)

        eb = base_e + NCHUNK * EC
        pltpu.sync_copy(ea_s.at[pl.ds(eb, ETAIL)], ixts)
        pltpu.sync_copy(ea_d.at[pl.ds(eb, ETAIL)], ixtd)
        pltpu.sync_copy(hv.at[ixts], rows0.at[pl.ds(0, ETAIL), :])
        pltpu.sync_copy(rows0.at[pl.ds(0, ETAIL), :], agg.at[ixtd], add=True)
        plsc.subcore_barrier()

        for j in range(RPT // WB):
            pltpu.async_copy(agg.at[pl.ds(rbase + j * WB, WB), :],
                             out.at[c, v, pl.ds(rbase + j * WB, WB), :], sw)
        pltpu.async_copy(agg.at[pl.ds(rbase + (RPT // WB) * WB, RPT % WB), :],
                         out.at[c, v,
                                pl.ds(rbase + (RPT // WB) * WB, RPT % WB), :],
                         sw)
        for j in range(RPT // WB):
            pltpu.make_async_copy(agg.at[pl.ds(rbase, WB), :],
                                  out.at[c, v, pl.ds(rbase, WB), :],
                                  sw).wait()
        pltpu.make_async_copy(agg.at[pl.ds(rbase, RPT % WB), :],
                              out.at[c, v, pl.ds(rbase, RPT % WB), :],
                              sw).wait()


_agg_call = functools.partial(
    pl.kernel,
    out_type=jax.ShapeDtypeStruct((NC, NV, NPAD, D), jnp.float32),
    mesh=_MESH,
    scratch_types=[
        pltpu.VMEM((EC, D), jnp.float32),
        pltpu.VMEM((EC, D), jnp.float32),
        pltpu.VMEM((EC,), jnp.int32),
        pltpu.VMEM((EC,), jnp.int32),
        pltpu.VMEM((EC,), jnp.int32),
        pltpu.VMEM((EC,), jnp.int32),
        pltpu.VMEM((EC,), jnp.int32),
        pltpu.VMEM((EC,), jnp.int32),
        pltpu.VMEM((EC,), jnp.int32),
        pltpu.VMEM((EC,), jnp.int32),
        pltpu.VMEM((ETAIL,), jnp.int32),
        pltpu.VMEM((ETAIL,), jnp.int32),
        pltpu.VMEM_SHARED((NPAD, D), jnp.float32),
        pltpu.SemaphoreType.DMA,
        pltpu.SemaphoreType.DMA,
        pltpu.SemaphoreType.DMA,
        pltpu.SemaphoreType.DMA,
        pltpu.SemaphoreType.DMA,
        pltpu.SemaphoreType.DMA,
        pltpu.SemaphoreType.DMA,
        pltpu.SemaphoreType.DMA,
        pltpu.SemaphoreType.DMA,
    ],
)(_agg_body)


def _scales_body(p_ref, o_ref):
    dg = jnp.sum(p_ref[...], axis=0)
    o_ref[...] = lax.rsqrt(jnp.maximum(dg, 1.0))


def _tcb_body(x_ref, so_ref, w_ref, o_ref):
    o_ref[0] = jnp.dot(x_ref[0] * so_ref[0], w_ref[...],
                       preferred_element_type=jnp.float32)


def _tcd_body(p_ref, si_ref, so_ref, b_ref, w_ref, o_ref):
    agg = p_ref[0, 0] + p_ref[1, 0]
    z = agg * si_ref[0] + b_ref[...]
    act = jnp.maximum(z, 0.0)
    o_ref[0] = jnp.dot(act * so_ref[0], w_ref[...],
                       preferred_element_type=jnp.float32)


def _tcf_body(p_ref, si_ref, b_ref, o_ref):
    o_ref[0] = (p_ref[0, 0] + p_ref[1, 0]) * si_ref[0] + b_ref[...]


_GRID = (NV, NPAD // BR)
_spec_x = pl.BlockSpec((1, BR, D), lambda v, i: (v, i, 0))
_spec_s = pl.BlockSpec((1, BR, 1), lambda v, i: (v, i, 0))
_spec_w = pl.BlockSpec((D, D), lambda v, i: (0, 0))
_spec_b = pl.BlockSpec((1, D), lambda v, i: (0, 0))
_spec_p = pl.BlockSpec((NC, 1, BR, D), lambda v, i: (0, v, i, 0))
_out_sds = jax.ShapeDtypeStruct((NV, NPAD, D), jnp.float32)


def _scales_call(p):
    return pl.pallas_call(
        _scales_body,
        out_shape=jax.ShapeDtypeStruct((6, NPAD), jnp.float32),
    )(p)


def _tcb_call(x, so, w):
    return pl.pallas_call(
        _tcb_body, grid=_GRID,
        in_specs=[_spec_x, _spec_s, _spec_w],
        out_specs=_spec_x, out_shape=_out_sds,
    )(x, so, w)


def _tcd_call(p, si, so, b, w):
    return pl.pallas_call(
        _tcd_body, grid=_GRID,
        in_specs=[_spec_p, _spec_s, _spec_s, _spec_b, _spec_w],
        out_specs=_spec_x, out_shape=_out_sds,
    )(p, si, so, b, w)


def _tcf_call(p, si, b):
    return pl.pallas_call(
        _tcf_body, grid=_GRID,
        in_specs=[_spec_p, _spec_s, _spec_b],
        out_specs=_spec_x, out_shape=_out_sds,
    )(p, si, b)


def kernel(feat1, feat2, feat, edge_index1, edge_index2, edge_index,
           W1, b1, W2, b2):
    feats = jnp.stack([feat1, feat2, feat])
    featsp = jnp.zeros((NV, NPAD, D), jnp.float32).at[:, :N, :].set(feats)
    e1 = edge_index1.astype(jnp.int32)
    e2 = edge_index2.astype(jnp.int32)
    e3 = edge_index.astype(jnp.int32)
    erows = (e1[0], e2[0], e3[0], e1[1], e2[1], e3[1])

    pdeg = _deg_call(*erows)
    scales = _scales_call(pdeg)
    s_out = scales[0:3].reshape(NV, NPAD, 1)
    s_in = scales[3:6].reshape(NV, NPAD, 1)

    h1 = _tcb_call(featsp, s_out, W1)
    p1 = _agg_call(h1, *erows)
    h2 = _tcd_call(p1, s_in, s_out, b1.reshape(1, D), W2)
    p2 = _agg_call(h2, *erows)
    z3 = _tcf_call(p2, s_in, b2.reshape(1, D))
    return (z3[0, :N], z3[1, :N], z3[2, :N])


# speedup vs baseline: 6.2151x; 1.1033x over previous
import functools

import jax
import jax.numpy as jnp
from jax import lax
from jax.experimental import pallas as pl
from jax.experimental.pallas import tpu as pltpu
from jax.experimental.pallas import tpu_sc as plsc

N = 10000
NPAD = 10112
D = 128
E = 320000
NV = 3
NC = 2
NS = 16
NW = NC * NS
RPT = NPAD // NS
EPT = E // NW
NH = NPAD // 2
CD = 640
NCD = EPT // CD
CDT = EPT - NCD * CD
EC = 64
NCHUNK = EPT // EC
ETAIL = EPT - NCHUNK * EC
WB = 64
BR = 1264

_MESH = plsc.VectorSubcoreMesh(
    core_axis_name="c", subcore_axis_name="s", num_cores=NC, num_subcores=NS
)


def _deg_body(es0, es1, es2, ed0, ed1, ed2, out, hist, idb0, idb1, sd0, sd1):
    earr = [es0, es1, es2, ed0, ed1, ed2]
    idb = [idb0, idb1]
    sd = [sd0, sd1]
    c = lax.axis_index("c")
    s = lax.axis_index("s")
    wid = c * NS + s
    base_e = wid * EPT
    zero16 = jnp.zeros((16,), jnp.float32)

    for a in range(6):
        ea = earr[a]

        def istart(k, b):
            n = CD if k < NCD else CDT
            pltpu.async_copy(ea.at[pl.ds(base_e + k * CD, n)],
                             idb[b].at[pl.ds(0, n)], sd[b])

        def iwait(k, b):
            n = CD if k < NCD else CDT
            pltpu.make_async_copy(ea.at[pl.ds(base_e, n)],
                                  idb[b].at[pl.ds(0, n)], sd[b]).wait()

        for half in range(2):
            lo = half * NH

            def zloop(r, _):
                hist[pl.ds(r * 16, 16)] = zero16
                return 0
            lax.fori_loop(0, NH // 16, zloop, 0)

            istart(0, 0)
            for k in range(NCD + 1):
                b = k % 2
                iwait(k, b)
                if k + 1 <= NCD:
                    istart(k + 1, (k + 1) % 2)
                nvec = (CD if k < NCD else CDT) // 16

                def vec(j, _):
                    vi = idb[b][pl.ds(j * 16, 16)]
                    m = (vi >= lo) & (vi < lo + NH)
                    cnt, lastm = plsc.scan_count(vi, mask=m)
                    li = jnp.where(m, vi - lo, 0)
                    plsc.addupdate_scatter(hist, [li],
                                           cnt.astype(jnp.float32),
                                           mask=lastm)
                    return 0
                lax.fori_loop(0, nvec, vec, 0)

            pltpu.sync_copy(hist,
                            out.at[pl.ds((wid * 6 + a) * NPAD + lo, NH)])


_deg_call = functools.partial(
    pl.kernel,
    out_type=jax.ShapeDtypeStruct((NW * 6 * NPAD,), jnp.float32),
    mesh=_MESH,
    scratch_types=[
        pltpu.VMEM((NH,), jnp.float32),
        pltpu.VMEM((CD,), jnp.int32),
        pltpu.VMEM((CD,), jnp.int32),
        pltpu.SemaphoreType.DMA,
        pltpu.SemaphoreType.DMA,
    ],
    compiler_params=pltpu.CompilerParams(needs_layout_passes=False),
)(_deg_body)


def _agg_body(hs, es0, es1, es2, ed0, ed1, ed2, out,
              rows0, rows1, ixs0, ixs1, ixs2, ixs3, ixd0, ixd1, ixd2, ixd3,
              ixts, ixtd, agg,
              sg0, sg1, ss0, ss1, si0, si1, si2, si3, sw):
    esrc = [es0, es1, es2]
    edst = [ed0, ed1, ed2]
    rows = [rows0, rows1]
    ixs = [ixs0, ixs1, ixs2, ixs3]
    ixd = [ixd0, ixd1, ixd2, ixd3]
    sg = [sg0, sg1]
    ss = [ss0, ss1]
    si = [si0, si1, si2, si3]
    c = lax.axis_index("c")
    s = lax.axis_index("s")
    base_e = (c * NS + s) * EPT
    rbase = s * RPT
    zero16 = jnp.zeros((16,), jnp.float32)

    for v in range(NV):
        ea_s = esrc[v]
        ea_d = edst[v]
        hv = hs.at[v]

        def istart(ci, sl):
            pltpu.async_copy(ea_s.at[pl.ds(base_e + ci * EC, EC)],
                             ixs[sl], si[sl])
            pltpu.async_copy(ea_d.at[pl.ds(base_e + ci * EC, EC)],
                             ixd[sl], si[sl])

        def iwait(sl):
            pltpu.make_async_copy(ea_s.at[pl.ds(base_e, EC)],
                                  ixs[sl], si[sl]).wait()
            pltpu.make_async_copy(ea_d.at[pl.ds(base_e, EC)],
                                  ixd[sl], si[sl]).wait()

        def gstart(sl, b):
            pltpu.async_copy(hv.at[ixs[sl]], rows[b], sg[b])

        def gwait(sl, b):
            pltpu.make_async_copy(hv.at[ixs[sl]], rows[b], sg[b]).wait()

        def sstart(sl, b):
            pltpu.async_copy(rows[b], agg.at[ixd[sl]], ss[b], add=True)

        def swait(sl, b):
            pltpu.make_async_copy(rows[b], agg.at[ixd[sl]], ss[b]).wait()

        def zfill(r, _):
            for j in range(D // 16):
                rows0[r, pl.ds(j * 16, 16)] = zero16
            return 0
        lax.fori_loop(0, WB, zfill, 0)
        for j in range(RPT // WB):
            pltpu.async_copy(rows0, agg.at[pl.ds(rbase + j * WB, WB), :], sw)
        pltpu.async_copy(rows0.at[pl.ds(0, RPT % WB), :],
                         agg.at[pl.ds(rbase + (RPT // WB) * WB, RPT % WB), :],
                         sw)
        for j in range(RPT // WB):
            pltpu.make_async_copy(rows0,
                                  agg.at[pl.ds(rbase, WB), :], sw).wait()
        pltpu.make_async_copy(rows0.at[pl.ds(0, RPT % WB), :],
                              agg.at[pl.ds(rbase, RPT % WB), :], sw).wait()
        plsc.subcore_barrier()

        istart(0, 0)
        istart(1, 1)
        istart(2, 2)
        iwait(0)
        gstart(0, 0)

        def quad(i4, _):
            i0 = i4 * 4
            for b in range(4):
                i = i0 + b
                rb = b % 2
                gwait(b, rb)
                sstart(b, rb)

                @pl.when(i + 1 < NCHUNK)
                def _():
                    iwait((b + 1) % 4)

                    @pl.when(i >= 1)
                    def _():
                        swait((b + 3) % 4, (b + 1) % 2)
                    gstart((b + 1) % 4, (b + 1) % 2)

                    @pl.when(i + 3 < NCHUNK)
                    def _():
                        istart(i + 3, (b + 3) % 4)
            return 0
        lax.fori_loop(0, NCHUNK // 4, quad, 0)
        swait(2, 0)
        swait(3, 1)

        eb = base_e + NCHUNK * EC
        pltpu.sync_copy(ea_s.at[pl.ds(eb, ETAIL)], ixts)
        pltpu.sync_copy(ea_d.at[pl.ds(eb, ETAIL)], ixtd)
        pltpu.sync_copy(hv.at[ixts], rows0.at[pl.ds(0, ETAIL), :])
        pltpu.sync_copy(rows0.at[pl.ds(0, ETAIL), :], agg.at[ixtd], add=True)
        plsc.subcore_barrier()

        for j in range(RPT // WB):
            pltpu.async_copy(agg.at[pl.ds(rbase + j * WB, WB), :],
                             out.at[c, v, pl.ds(rbase + j * WB, WB), :], sw)
        pltpu.async_copy(agg.at[pl.ds(rbase + (RPT // WB) * WB, RPT % WB), :],
                         out.at[c, v,
                                pl.ds(rbase + (RPT // WB) * WB, RPT % WB), :],
                         sw)
        for j in range(RPT // WB):
            pltpu.make_async_copy(agg.at[pl.ds(rbase, WB), :],
                                  out.at[c, v, pl.ds(rbase, WB), :],
                                  sw).wait()
        pltpu.make_async_copy(agg.at[pl.ds(rbase, RPT % WB), :],
                              out.at[c, v, pl.ds(rbase, RPT % WB), :],
                              sw).wait()


_agg_call = functools.partial(
    pl.kernel,
    out_type=jax.ShapeDtypeStruct((NC, NV, NPAD, D), jnp.float32),
    mesh=_MESH,
    scratch_types=[
        pltpu.VMEM((EC, D), jnp.float32),
        pltpu.VMEM((EC, D), jnp.float32),
        pltpu.VMEM((EC,), jnp.int32),
        pltpu.VMEM((EC,), jnp.int32),
        pltpu.VMEM((EC,), jnp.int32),
        pltpu.VMEM((EC,), jnp.int32),
        pltpu.VMEM((EC,), jnp.int32),
        pltpu.VMEM((EC,), jnp.int32),
        pltpu.VMEM((EC,), jnp.int32),
        pltpu.VMEM((EC,), jnp.int32),
        pltpu.VMEM((ETAIL,), jnp.int32),
        pltpu.VMEM((ETAIL,), jnp.int32),
        pltpu.VMEM_SHARED((NPAD, D), jnp.float32),
        pltpu.SemaphoreType.DMA,
        pltpu.SemaphoreType.DMA,
        pltpu.SemaphoreType.DMA,
        pltpu.SemaphoreType.DMA,
        pltpu.SemaphoreType.DMA,
        pltpu.SemaphoreType.DMA,
        pltpu.SemaphoreType.DMA,
        pltpu.SemaphoreType.DMA,
        pltpu.SemaphoreType.DMA,
    ],
)(_agg_body)


def _scales_body(p_ref, o_ref):
    dg = jnp.sum(p_ref[...], axis=0)
    o_ref[...] = lax.rsqrt(jnp.maximum(dg, 1.0))


def _tcb_body(x_ref, so_ref, w_ref, o_ref):
    o_ref[0] = jnp.dot(x_ref[0] * so_ref[0], w_ref[...],
                       preferred_element_type=jnp.float32)


def _tcd_body(p_ref, si_ref, so_ref, b_ref, w_ref, o_ref):
    agg = p_ref[0, 0] + p_ref[1, 0]
    z = agg * si_ref[0] + b_ref[...]
    act = jnp.maximum(z, 0.0)
    o_ref[0] = jnp.dot(act * so_ref[0], w_ref[...],
                       preferred_element_type=jnp.float32)


def _tcf_body(p_ref, si_ref, b_ref, o_ref):
    o_ref[0] = (p_ref[0, 0] + p_ref[1, 0]) * si_ref[0] + b_ref[...]


_GRID = (NV, NPAD // BR)
_spec_x = pl.BlockSpec((1, BR, D), lambda v, i: (v, i, 0))
_spec_s = pl.BlockSpec((1, BR, 1), lambda v, i: (v, i, 0))
_spec_w = pl.BlockSpec((D, D), lambda v, i: (0, 0))
_spec_b = pl.BlockSpec((1, D), lambda v, i: (0, 0))
_spec_p = pl.BlockSpec((NC, 1, BR, D), lambda v, i: (0, v, i, 0))
_out_sds = jax.ShapeDtypeStruct((NV, NPAD, D), jnp.float32)


def _scales_call(p):
    return pl.pallas_call(
        _scales_body,
        out_shape=jax.ShapeDtypeStruct((6, NPAD), jnp.float32),
    )(p)


def _tcb_call(x, so, w):
    return pl.pallas_call(
        _tcb_body, grid=_GRID,
        in_specs=[_spec_x, _spec_s, _spec_w],
        out_specs=_spec_x, out_shape=_out_sds,
    )(x, so, w)


def _tcd_call(p, si, so, b, w):
    return pl.pallas_call(
        _tcd_body, grid=_GRID,
        in_specs=[_spec_p, _spec_s, _spec_s, _spec_b, _spec_w],
        out_specs=_spec_x, out_shape=_out_sds,
    )(p, si, so, b, w)


def _tcf_call(p, si, b):
    return pl.pallas_call(
        _tcf_body, grid=_GRID,
        in_specs=[_spec_p, _spec_s, _spec_b],
        out_specs=_spec_x, out_shape=_out_sds,
    )(p, si, b)


def kernel(feat1, feat2, feat, edge_index1, edge_index2, edge_index,
           W1, b1, W2, b2):
    feats = jnp.stack([feat1, feat2, feat])
    featsp = jnp.zeros((NV, NPAD, D), jnp.float32).at[:, :N, :].set(feats)
    e1 = edge_index1.astype(jnp.int32)
    e2 = edge_index2.astype(jnp.int32)
    e3 = edge_index.astype(jnp.int32)
    erows = (e1[0], e2[0], e3[0], e1[1], e2[1], e3[1])

    pdeg = _deg_call(*erows).reshape(NW, 6, NPAD)
    scales = _scales_call(pdeg)
    s_out = scales[0:3].reshape(NV, NPAD, 1)
    s_in = scales[3:6].reshape(NV, NPAD, 1)

    h1 = _tcb_call(featsp, s_out, W1)
    p1 = _agg_call(h1, *erows)
    h2 = _tcd_call(p1, s_in, s_out, b1.reshape(1, D), W2)
    p2 = _agg_call(h2, *erows)
    z3 = _tcf_call(p2, s_in, b2.reshape(1, D))
    return (z3[0, :N], z3[1, :N], z3[2, :N])


# speedup vs baseline: 6.3795x; 1.0265x over previous
import functools

import jax
import jax.numpy as jnp
from jax import lax
from jax.experimental import pallas as pl
from jax.experimental.pallas import tpu as pltpu
from jax.experimental.pallas import tpu_sc as plsc

N = 10000
NPAD = 10112
D = 128
E = 320000
NV = 3
NC = 2
NS = 16
NW = NC * NS
RPT = NPAD // NS
EPT = E // NW
NH = NPAD // 2
CD = 640
NCD = EPT // CD
CDT = EPT - NCD * CD
EC = 64
NCHUNK = EPT // EC
ETAIL = EPT - NCHUNK * EC
WB = 64
BR = 1264

_MESH = plsc.VectorSubcoreMesh(
    core_axis_name="c", subcore_axis_name="s", num_cores=NC, num_subcores=NS
)


def _deg_body(es0, es1, es2, ed0, ed1, ed2, out, hist, idb0, idb1, sd0, sd1):
    earr = [es0, es1, es2, ed0, ed1, ed2]
    idb = [idb0, idb1]
    sd = [sd0, sd1]
    c = lax.axis_index("c")
    s = lax.axis_index("s")
    wid = c * NS + s
    base_e = wid * EPT
    zero16 = jnp.zeros((16,), jnp.float32)

    for a in range(6):
        ea = earr[a]

        def istart(k, b):
            n = CD if k < NCD else CDT
            pltpu.async_copy(ea.at[pl.ds(base_e + k * CD, n)],
                             idb[b].at[pl.ds(0, n)], sd[b])

        def iwait(k, b):
            n = CD if k < NCD else CDT
            pltpu.make_async_copy(ea.at[pl.ds(base_e, n)],
                                  idb[b].at[pl.ds(0, n)], sd[b]).wait()

        for half in range(2):
            lo = half * NH

            def zloop(r, _):
                hist[pl.ds(r * 16, 16)] = zero16
                return 0
            lax.fori_loop(0, NH // 16, zloop, 0)

            istart(0, 0)
            for k in range(NCD + 1):
                b = k % 2
                iwait(k, b)
                if k + 1 <= NCD:
                    istart(k + 1, (k + 1) % 2)
                nvec = (CD if k < NCD else CDT) // 16

                def vec(j, _):
                    vi = idb[b][pl.ds(j * 16, 16)]
                    m = (vi >= lo) & (vi < lo + NH)
                    cnt, lastm = plsc.scan_count(vi, mask=m)
                    li = jnp.where(m, vi - lo, 0)
                    plsc.addupdate_scatter(hist, [li],
                                           cnt.astype(jnp.float32),
                                           mask=lastm)
                    return 0
                lax.fori_loop(0, nvec, vec, 0)

            pltpu.sync_copy(hist,
                            out.at[pl.ds((wid * 6 + a) * NPAD + lo, NH)])


_deg_call = functools.partial(
    pl.kernel,
    out_type=jax.ShapeDtypeStruct((NW * 6 * NPAD,), jnp.float32),
    mesh=_MESH,
    scratch_types=[
        pltpu.VMEM((NH,), jnp.float32),
        pltpu.VMEM((CD,), jnp.int32),
        pltpu.VMEM((CD,), jnp.int32),
        pltpu.SemaphoreType.DMA,
        pltpu.SemaphoreType.DMA,
    ],
    compiler_params=pltpu.CompilerParams(needs_layout_passes=False),
)(_deg_body)


def _agg_body(h0, h1, h2, es0, es1, es2, ed0, ed1, ed2, out0, out1, out2,
              rows0, rows1, ixs0, ixs1, ixs2, ixs3, ixd0, ixd1, ixd2, ixd3,
              ixts, ixtd, agg,
              sg0, sg1, ss0, ss1, si0, si1, si2, si3, sw):
    harr = [h0, h1, h2]
    oarr = [out0, out1, out2]
    esrc = [es0, es1, es2]
    edst = [ed0, ed1, ed2]
    rows = [rows0, rows1]
    ixs = [ixs0, ixs1, ixs2, ixs3]
    ixd = [ixd0, ixd1, ixd2, ixd3]
    sg = [sg0, sg1]
    ss = [ss0, ss1]
    si = [si0, si1, si2, si3]
    c = lax.axis_index("c")
    s = lax.axis_index("s")
    base_e = (c * NS + s) * EPT
    rbase = s * RPT
    zero16 = jnp.zeros((16,), jnp.float32)

    for v in range(NV):
        ea_s = esrc[v]
        ea_d = edst[v]
        hv = harr[v]
        ov = oarr[v]

        def istart(ci, sl):
            pltpu.async_copy(ea_s.at[pl.ds(base_e + ci * EC, EC)],
                             ixs[sl], si[sl])
            pltpu.async_copy(ea_d.at[pl.ds(base_e + ci * EC, EC)],
                             ixd[sl], si[sl])

        def iwait(sl):
            pltpu.make_async_copy(ea_s.at[pl.ds(base_e, EC)],
                                  ixs[sl], si[sl]).wait()
            pltpu.make_async_copy(ea_d.at[pl.ds(base_e, EC)],
                                  ixd[sl], si[sl]).wait()

        def gstart(sl, b):
            pltpu.async_copy(hv.at[ixs[sl]], rows[b], sg[b])

        def gwait(sl, b):
            pltpu.make_async_copy(hv.at[ixs[sl]], rows[b], sg[b]).wait()

        def sstart(sl, b):
            pltpu.async_copy(rows[b], agg.at[ixd[sl]], ss[b], add=True)

        def swait(sl, b):
            pltpu.make_async_copy(rows[b], agg.at[ixd[sl]], ss[b]).wait()

        def zfill(r, _):
            for j in range(D // 16):
                rows0[r, pl.ds(j * 16, 16)] = zero16
            return 0
        lax.fori_loop(0, WB, zfill, 0)
        for j in range(RPT // WB):
            pltpu.async_copy(rows0, agg.at[pl.ds(rbase + j * WB, WB), :], sw)
        pltpu.async_copy(rows0.at[pl.ds(0, RPT % WB), :],
                         agg.at[pl.ds(rbase + (RPT // WB) * WB, RPT % WB), :],
                         sw)
        for j in range(RPT // WB):
            pltpu.make_async_copy(rows0,
                                  agg.at[pl.ds(rbase, WB), :], sw).wait()
        pltpu.make_async_copy(rows0.at[pl.ds(0, RPT % WB), :],
                              agg.at[pl.ds(rbase, RPT % WB), :], sw).wait()
        plsc.subcore_barrier()

        istart(0, 0)
        istart(1, 1)
        istart(2, 2)
        iwait(0)
        gstart(0, 0)

        def quad(i4, _):
            i0 = i4 * 4
            for b in range(4):
                i = i0 + b
                rb = b % 2
                gwait(b, rb)
                sstart(b, rb)

                @pl.when(i + 1 < NCHUNK)
                def _():
                    iwait((b + 1) % 4)

                    @pl.when(i >= 1)
                    def _():
                        swait((b + 3) % 4, (b + 1) % 2)
                    gstart((b + 1) % 4, (b + 1) % 2)

                    @pl.when(i + 3 < NCHUNK)
                    def _():
                        istart(i + 3, (b + 3) % 4)
            return 0
        lax.fori_loop(0, NCHUNK // 4, quad, 0)
        swait(2, 0)
        swait(3, 1)

        eb = base_e + NCHUNK * EC
        pltpu.sync_copy(ea_s.at[pl.ds(eb, ETAIL)], ixts)
        pltpu.sync_copy(ea_d.at[pl.ds(eb, ETAIL)], ixtd)
        pltpu.sync_copy(hv.at[ixts], rows0.at[pl.ds(0, ETAIL), :])
        pltpu.sync_copy(rows0.at[pl.ds(0, ETAIL), :], agg.at[ixtd], add=True)
        plsc.subcore_barrier()

        for j in range(RPT // WB):
            pltpu.async_copy(agg.at[pl.ds(rbase + j * WB, WB), :],
                             ov.at[c, pl.ds(rbase + j * WB, WB), :], sw)
        pltpu.async_copy(agg.at[pl.ds(rbase + (RPT // WB) * WB, RPT % WB), :],
                         ov.at[c,
                               pl.ds(rbase + (RPT // WB) * WB, RPT % WB), :],
                         sw)
        for j in range(RPT // WB):
            pltpu.make_async_copy(agg.at[pl.ds(rbase, WB), :],
                                  ov.at[c, pl.ds(rbase, WB), :],
                                  sw).wait()
        pltpu.make_async_copy(agg.at[pl.ds(rbase, RPT % WB), :],
                              ov.at[c, pl.ds(rbase, RPT % WB), :],
                              sw).wait()


_agg_call = functools.partial(
    pl.kernel,
    out_type=[jax.ShapeDtypeStruct((NC, NPAD, D), jnp.float32)] * 3,
    mesh=_MESH,
    scratch_types=[
        pltpu.VMEM((EC, D), jnp.float32),
        pltpu.VMEM((EC, D), jnp.float32),
        pltpu.VMEM((EC,), jnp.int32),
        pltpu.VMEM((EC,), jnp.int32),
        pltpu.VMEM((EC,), jnp.int32),
        pltpu.VMEM((EC,), jnp.int32),
        pltpu.VMEM((EC,), jnp.int32),
        pltpu.VMEM((EC,), jnp.int32),
        pltpu.VMEM((EC,), jnp.int32),
        pltpu.VMEM((EC,), jnp.int32),
        pltpu.VMEM((ETAIL,), jnp.int32),
        pltpu.VMEM((ETAIL,), jnp.int32),
        pltpu.VMEM_SHARED((NPAD, D), jnp.float32),
        pltpu.SemaphoreType.DMA,
        pltpu.SemaphoreType.DMA,
        pltpu.SemaphoreType.DMA,
        pltpu.SemaphoreType.DMA,
        pltpu.SemaphoreType.DMA,
        pltpu.SemaphoreType.DMA,
        pltpu.SemaphoreType.DMA,
        pltpu.SemaphoreType.DMA,
        pltpu.SemaphoreType.DMA,
    ],
)(_agg_body)


def _scales_body(p_ref, o_ref):
    dg = jnp.sum(p_ref[...], axis=0)
    o_ref[...] = lax.rsqrt(jnp.maximum(dg, 1.0))


def _tcb_body(x0, x1, x2, s0, s1, s2, w, o0, o1, o2):
    for x, sv, o in ((x0, s0, o0), (x1, s1, o1), (x2, s2, o2)):
        o[...] = jnp.dot(x[...] * sv[...], w[...],
                         preferred_element_type=jnp.float32)


def _tcd_body(p0, p1, p2, i0, i1, i2, s0, s1, s2, b, w, o0, o1, o2):
    for p, si, so, o in ((p0, i0, s0, o0), (p1, i1, s1, o1),
                         (p2, i2, s2, o2)):
        z = (p[0] + p[1]) * si[...] + b[...]
        act = jnp.maximum(z, 0.0)
        o[...] = jnp.dot(act * so[...], w[...],
                         preferred_element_type=jnp.float32)


def _tcf_body(p0, p1, p2, i0, i1, i2, b, o0, o1, o2):
    for p, si, o in ((p0, i0, o0), (p1, i1, o1), (p2, i2, o2)):
        o[...] = (p[0] + p[1]) * si[...] + b[...]


BR2 = 2000
_GRID = (N // BR2,)
_spec_x = pl.BlockSpec((BR2, D), lambda i: (i, 0))
_spec_s = pl.BlockSpec((BR2, 1), lambda i: (i, 0))
_spec_w = pl.BlockSpec((D, D), lambda i: (0, 0))
_spec_b = pl.BlockSpec((1, D), lambda i: (0, 0))
_spec_p = pl.BlockSpec((NC, BR2, D), lambda i: (0, i, 0))
_h_sds = jax.ShapeDtypeStruct((N, D), jnp.float32)


def _scales_call(p):
    return pl.pallas_call(
        _scales_body,
        out_shape=jax.ShapeDtypeStruct((6, NPAD), jnp.float32),
    )(p)


def _tcb_call(xs, so, w):
    return pl.pallas_call(
        _tcb_body, grid=_GRID,
        in_specs=[_spec_x] * 3 + [_spec_s] * 3 + [_spec_w],
        out_specs=[_spec_x] * 3, out_shape=[_h_sds] * 3,
    )(*xs, *so, w)


def _tcd_call(ps, si, so, b, w):
    return pl.pallas_call(
        _tcd_body, grid=_GRID,
        in_specs=[_spec_p] * 3 + [_spec_s] * 6 + [_spec_b, _spec_w],
        out_specs=[_spec_x] * 3, out_shape=[_h_sds] * 3,
    )(*ps, *si, *so, b, w)


def _tcf_call(ps, si, b):
    return pl.pallas_call(
        _tcf_body, grid=_GRID,
        in_specs=[_spec_p] * 3 + [_spec_s] * 3 + [_spec_b],
        out_specs=[_spec_x] * 3, out_shape=[_h_sds] * 3,
    )(*ps, *si, b)


def kernel(feat1, feat2, feat, edge_index1, edge_index2, edge_index,
           W1, b1, W2, b2):
    e1 = edge_index1.astype(jnp.int32)
    e2 = edge_index2.astype(jnp.int32)
    e3 = edge_index.astype(jnp.int32)
    erows = (e1[0], e2[0], e3[0], e1[1], e2[1], e3[1])

    pdeg = _deg_call(*erows).reshape(NW, 6, NPAD)
    scales = _scales_call(pdeg)
    so = [scales[v, :N].reshape(N, 1) for v in range(3)]
    si = [scales[3 + v, :N].reshape(N, 1) for v in range(3)]

    h1 = _tcb_call((feat1, feat2, feat), so, W1)
    p1 = _agg_call(*h1, *erows)
    h2 = _tcd_call(p1, si, so, b1.reshape(1, D), W2)
    p2 = _agg_call(*h2, *erows)
    return tuple(_tcf_call(p2, si, b2.reshape(1, D)))


# speedup vs baseline: 6.4201x; 1.0064x over previous
import functools

import jax
import jax.numpy as jnp
from jax import lax
from jax.experimental import pallas as pl
from jax.experimental.pallas import tpu as pltpu
from jax.experimental.pallas import tpu_sc as plsc

N = 10000
NPAD = 10112
D = 128
E = 320000
NV = 3
NC = 2
NS = 16
NW = NC * NS
RPT = NPAD // NS
EPT = E // NW
NH = NPAD // 2
CD = 640
NCD = EPT // CD
CDT = EPT - NCD * CD
EC = 64
NCHUNK = EPT // EC
ETAIL = EPT - NCHUNK * EC
WB = 64
BR = 1264

_MESH = plsc.VectorSubcoreMesh(
    core_axis_name="c", subcore_axis_name="s", num_cores=NC, num_subcores=NS
)


def _deg_body(es0, es1, es2, ed0, ed1, ed2, out, hist, idb0, idb1, sd0, sd1):
    earr = [es0, es1, es2, ed0, ed1, ed2]
    idb = [idb0, idb1]
    sd = [sd0, sd1]
    c = lax.axis_index("c")
    s = lax.axis_index("s")
    wid = c * NS + s
    base_e = wid * EPT
    zero16 = jnp.zeros((16,), jnp.float32)

    for a in range(6):
        ea = earr[a]

        def istart(k, b):
            n = CD if k < NCD else CDT
            pltpu.async_copy(ea.at[pl.ds(base_e + k * CD, n)],
                             idb[b].at[pl.ds(0, n)], sd[b])

        def iwait(k, b):
            n = CD if k < NCD else CDT
            pltpu.make_async_copy(ea.at[pl.ds(base_e, n)],
                                  idb[b].at[pl.ds(0, n)], sd[b]).wait()

        for half in range(2):
            lo = half * NH

            def zloop(r, _):
                hist[pl.ds(r * 16, 16)] = zero16
                return 0
            lax.fori_loop(0, NH // 16, zloop, 0)

            istart(0, 0)
            for k in range(NCD + 1):
                b = k % 2
                iwait(k, b)
                if k + 1 <= NCD:
                    istart(k + 1, (k + 1) % 2)
                nvec = (CD if k < NCD else CDT) // 16

                def vec(j, _):
                    vi = idb[b][pl.ds(j * 16, 16)]
                    m = (vi >= lo) & (vi < lo + NH)
                    cnt, lastm = plsc.scan_count(vi, mask=m)
                    li = jnp.where(m, vi - lo, 0)
                    plsc.addupdate_scatter(hist, [li],
                                           cnt.astype(jnp.float32),
                                           mask=lastm)
                    return 0
                lax.fori_loop(0, nvec, vec, 0)

            pltpu.sync_copy(hist,
                            out.at[pl.ds((wid * 6 + a) * NPAD + lo, NH)])


_deg_call = functools.partial(
    pl.kernel,
    out_type=jax.ShapeDtypeStruct((NW * 6 * NPAD,), jnp.float32),
    mesh=_MESH,
    scratch_types=[
        pltpu.VMEM((NH,), jnp.float32),
        pltpu.VMEM((CD,), jnp.int32),
        pltpu.VMEM((CD,), jnp.int32),
        pltpu.SemaphoreType.DMA,
        pltpu.SemaphoreType.DMA,
    ],
    compiler_params=pltpu.CompilerParams(needs_layout_passes=False),
)(_deg_body)


def _agg_body(hv, ea_s, ea_d, ov,
              rows0, rows1, ixs0, ixs1, ixs2, ixs3, ixd0, ixd1, ixd2, ixd3,
              ixts, ixtd, agg,
              sg0, sg1, ss0, ss1, si0, si1, si2, si3, sw):
    rows = [rows0, rows1]
    ixs = [ixs0, ixs1, ixs2, ixs3]
    ixd = [ixd0, ixd1, ixd2, ixd3]
    sg = [sg0, sg1]
    ss = [ss0, ss1]
    si = [si0, si1, si2, si3]
    c = lax.axis_index("c")
    s = lax.axis_index("s")
    base_e = (c * NS + s) * EPT
    rbase = s * RPT
    zero16 = jnp.zeros((16,), jnp.float32)

    if True:
        def istart(ci, sl):
            pltpu.async_copy(ea_s.at[pl.ds(base_e + ci * EC, EC)],
                             ixs[sl], si[sl])
            pltpu.async_copy(ea_d.at[pl.ds(base_e + ci * EC, EC)],
                             ixd[sl], si[sl])

        def iwait(sl):
            pltpu.make_async_copy(ea_s.at[pl.ds(base_e, EC)],
                                  ixs[sl], si[sl]).wait()
            pltpu.make_async_copy(ea_d.at[pl.ds(base_e, EC)],
                                  ixd[sl], si[sl]).wait()

        def gstart(sl, b):
            pltpu.async_copy(hv.at[ixs[sl]], rows[b], sg[b])

        def gwait(sl, b):
            pltpu.make_async_copy(hv.at[ixs[sl]], rows[b], sg[b]).wait()

        def sstart(sl, b):
            pltpu.async_copy(rows[b], agg.at[ixd[sl]], ss[b], add=True)

        def swait(sl, b):
            pltpu.make_async_copy(rows[b], agg.at[ixd[sl]], ss[b]).wait()

        def zfill(r, _):
            for j in range(D // 16):
                rows0[r, pl.ds(j * 16, 16)] = zero16
            return 0
        lax.fori_loop(0, WB, zfill, 0)
        for j in range(RPT // WB):
            pltpu.async_copy(rows0, agg.at[pl.ds(rbase + j * WB, WB), :], sw)
        pltpu.async_copy(rows0.at[pl.ds(0, RPT % WB), :],
                         agg.at[pl.ds(rbase + (RPT // WB) * WB, RPT % WB), :],
                         sw)
        for j in range(RPT // WB):
            pltpu.make_async_copy(rows0,
                                  agg.at[pl.ds(rbase, WB), :], sw).wait()
        pltpu.make_async_copy(rows0.at[pl.ds(0, RPT % WB), :],
                              agg.at[pl.ds(rbase, RPT % WB), :], sw).wait()
        plsc.subcore_barrier()

        istart(0, 0)
        istart(1, 1)
        istart(2, 2)
        iwait(0)
        gstart(0, 0)

        def quad(i4, _):
            i0 = i4 * 4
            for b in range(4):
                i = i0 + b
                rb = b % 2
                gwait(b, rb)
                sstart(b, rb)

                @pl.when(i + 1 < NCHUNK)
                def _():
                    iwait((b + 1) % 4)

                    @pl.when(i >= 1)
                    def _():
                        swait((b + 3) % 4, (b + 1) % 2)
                    gstart((b + 1) % 4, (b + 1) % 2)

                    @pl.when(i + 3 < NCHUNK)
                    def _():
                        istart(i + 3, (b + 3) % 4)
            return 0
        lax.fori_loop(0, NCHUNK // 4, quad, 0)
        swait(2, 0)
        swait(3, 1)

        eb = base_e + NCHUNK * EC
        pltpu.sync_copy(ea_s.at[pl.ds(eb, ETAIL)], ixts)
        pltpu.sync_copy(ea_d.at[pl.ds(eb, ETAIL)], ixtd)
        pltpu.sync_copy(hv.at[ixts], rows0.at[pl.ds(0, ETAIL), :])
        pltpu.sync_copy(rows0.at[pl.ds(0, ETAIL), :], agg.at[ixtd], add=True)
        plsc.subcore_barrier()

        for j in range(RPT // WB):
            pltpu.async_copy(agg.at[pl.ds(rbase + j * WB, WB), :],
                             ov.at[c, pl.ds(rbase + j * WB, WB), :], sw)
        pltpu.async_copy(agg.at[pl.ds(rbase + (RPT // WB) * WB, RPT % WB), :],
                         ov.at[c,
                               pl.ds(rbase + (RPT // WB) * WB, RPT % WB), :],
                         sw)
        for j in range(RPT // WB):
            pltpu.make_async_copy(agg.at[pl.ds(rbase, WB), :],
                                  ov.at[c, pl.ds(rbase, WB), :],
                                  sw).wait()
        pltpu.make_async_copy(agg.at[pl.ds(rbase, RPT % WB), :],
                              ov.at[c, pl.ds(rbase, RPT % WB), :],
                              sw).wait()


_agg_call = functools.partial(
    pl.kernel,
    out_type=jax.ShapeDtypeStruct((NC, NPAD, D), jnp.float32),
    mesh=_MESH,
    scratch_types=[
        pltpu.VMEM((EC, D), jnp.float32),
        pltpu.VMEM((EC, D), jnp.float32),
        pltpu.VMEM((EC,), jnp.int32),
        pltpu.VMEM((EC,), jnp.int32),
        pltpu.VMEM((EC,), jnp.int32),
        pltpu.VMEM((EC,), jnp.int32),
        pltpu.VMEM((EC,), jnp.int32),
        pltpu.VMEM((EC,), jnp.int32),
        pltpu.VMEM((EC,), jnp.int32),
        pltpu.VMEM((EC,), jnp.int32),
        pltpu.VMEM((ETAIL,), jnp.int32),
        pltpu.VMEM((ETAIL,), jnp.int32),
        pltpu.VMEM_SHARED((NPAD, D), jnp.float32),
        pltpu.SemaphoreType.DMA,
        pltpu.SemaphoreType.DMA,
        pltpu.SemaphoreType.DMA,
        pltpu.SemaphoreType.DMA,
        pltpu.SemaphoreType.DMA,
        pltpu.SemaphoreType.DMA,
        pltpu.SemaphoreType.DMA,
        pltpu.SemaphoreType.DMA,
        pltpu.SemaphoreType.DMA,
    ],
)(_agg_body)


def _scales_body(p_ref, o_ref):
    dg = jnp.sum(p_ref[...], axis=0)
    o_ref[...] = lax.rsqrt(jnp.maximum(dg, 1.0))


def _tcb_body(x0, x1, x2, s0, s1, s2, w, o0, o1, o2):
    for x, sv, o in ((x0, s0, o0), (x1, s1, o1), (x2, s2, o2)):
        o[...] = jnp.dot(x[...] * sv[...], w[...],
                         preferred_element_type=jnp.float32)


def _tcd_body(p, si, so, b, w, o):
    z = (p[0] + p[1]) * si[...] + b[...]
    act = jnp.maximum(z, 0.0)
    o[...] = jnp.dot(act * so[...], w[...],
                     preferred_element_type=jnp.float32)


def _tcf_body(p, si, b, o):
    o[...] = (p[0] + p[1]) * si[...] + b[...]


BR2 = 2000
_GRID = (N // BR2,)
_spec_x = pl.BlockSpec((BR2, D), lambda i: (i, 0))
_spec_s = pl.BlockSpec((BR2, 1), lambda i: (i, 0))
_spec_w = pl.BlockSpec((D, D), lambda i: (0, 0))
_spec_b = pl.BlockSpec((1, D), lambda i: (0, 0))
_spec_p = pl.BlockSpec((NC, BR2, D), lambda i: (0, i, 0))
_h_sds = jax.ShapeDtypeStruct((N, D), jnp.float32)


def _scales_call(p):
    return pl.pallas_call(
        _scales_body,
        out_shape=jax.ShapeDtypeStruct((6, NPAD), jnp.float32),
    )(p)


def _tcb_call(xs, so, w):
    return pl.pallas_call(
        _tcb_body, grid=_GRID,
        in_specs=[_spec_x] * 3 + [_spec_s] * 3 + [_spec_w],
        out_specs=[_spec_x] * 3, out_shape=[_h_sds] * 3,
    )(*xs, *so, w)


def _tcd_call(p, si, so, b, w):
    return pl.pallas_call(
        _tcd_body, grid=_GRID,
        in_specs=[_spec_p, _spec_s, _spec_s, _spec_b, _spec_w],
        out_specs=_spec_x, out_shape=_h_sds,
    )(p, si, so, b, w)


def _tcf_call(p, si, b):
    return pl.pallas_call(
        _tcf_body, grid=_GRID,
        in_specs=[_spec_p, _spec_s, _spec_b],
        out_specs=_spec_x, out_shape=_h_sds,
    )(p, si, b)


def kernel(feat1, feat2, feat, edge_index1, edge_index2, edge_index,
           W1, b1, W2, b2):
    e1 = edge_index1.astype(jnp.int32)
    e2 = edge_index2.astype(jnp.int32)
    e3 = edge_index.astype(jnp.int32)
    erows = (e1[0], e2[0], e3[0], e1[1], e2[1], e3[1])

    pdeg = _deg_call(*erows).reshape(NW, 6, NPAD)
    scales = _scales_call(pdeg)
    so = [scales[v, :N].reshape(N, 1) for v in range(3)]
    si = [scales[3 + v, :N].reshape(N, 1) for v in range(3)]

    b1r = b1.reshape(1, D)
    b2r = b2.reshape(1, D)
    h1 = _tcb_call((feat1, feat2, feat), so, W1)
    p1 = [_agg_call(h1[v], erows[v], erows[3 + v]) for v in range(NV)]
    h2 = [_tcd_call(p1[v], si[v], so[v], b1r, W2) for v in range(NV)]
    p2 = [_agg_call(h2[v], erows[v], erows[3 + v]) for v in range(NV)]
    return tuple(_tcf_call(p2[v], si[v], b2r) for v in range(NV))
